# baseline (speedup 1.0000x reference)
import functools

import jax
import jax.numpy as jnp
from jax.experimental import pallas as pl
from jax.experimental.pallas import tpu as pltpu


def _yunet_body(x_ref, wdn_ref, bdn_ref, wll_ref, bll_ref,
                wbb_ref, bbb_ref, whd_ref, bhd_ref,
                cls_ref, bbox_ref, obj_ref, kps_ref,
                f_ref, col_ref, *, H, W, HWB):
    HW = H * W
    j = pl.program_id(1)

    @pl.when(j == 0)
    def _convs():
        lane = jax.lax.broadcasted_iota(jnp.int32, (1, HW), 1)
        row = lane // W
        col = lane % W

        taps = []
        for di in range(3):
            for dj in range(3):
                s = (di - 1) * W + (dj - 1)
                amount = (-s) % HW
                valid = ((row + (di - 1) >= 0) & (row + (di - 1) < H) &
                         (col + (dj - 1) >= 0) & (col + (dj - 1) < W))
                taps.append((amount, valid))

        def conv3x3_relu(x, w_ref, b_ref):
            cin = x.shape[0]
            for t, (amount, valid) in enumerate(taps):
                shifted = x if amount == 0 else pltpu.roll(x, amount, axis=1)
                col_ref[t * cin:(t + 1) * cin, :] = jnp.where(valid, shifted, 0.0)
            y = jnp.dot(w_ref[...], col_ref[...],
                        preferred_element_type=jnp.float32) + b_ref[...]
            return jnp.maximum(y, 0.0)

        x = x_ref[0]
        x = conv3x3_relu(x, wdn_ref, bdn_ref)
        x = conv3x3_relu(x, wll_ref, bll_ref)
        f_ref[...] = conv3x3_relu(x, wbb_ref, bbb_ref)

    fc = f_ref[:, pl.ds(j * HWB, HWB)]
    yt = jax.lax.dot_general(fc, whd_ref[...],
                             dimension_numbers=(((0,), (1,)), ((), ())),
                             preferred_element_type=jnp.float32) + bhd_ref[...]
    cls_ref[0] = yt[:, 0:1]
    bbox_ref[0] = yt[:, 1:5]
    obj_ref[0] = yt[:, 5:6]
    kps_ref[0] = yt[:, 6:16]


def kernel(img, dn_w, dn_b, lle_w, lle_b, bb_w, bb_b, hd_w, hd_b):
    n, c, h, w = img.shape
    hw = h * w
    x = img.astype(jnp.float32).reshape(n, c, hw)

    def conv_w(wt):
        return jnp.transpose(wt, (0, 2, 3, 1)).reshape(wt.shape[0], -1)

    chunks = 4 if hw % (4 * 128) == 0 else 1
    hwb = hw // chunks

    operands = (
        x,
        conv_w(dn_w), dn_b.reshape(-1, 1),
        conv_w(lle_w), lle_b.reshape(-1, 1),
        conv_w(bb_w), bb_b.reshape(-1, 1),
        hd_w, hd_b.reshape(1, -1),
    )
    in_specs = [pl.BlockSpec((1, c, hw), lambda i, j: (i, 0, 0))]
    in_specs += [pl.BlockSpec(op.shape, lambda i, j: (0, 0))
                 for op in operands[1:]]

    out_shapes = tuple(jax.ShapeDtypeStruct((n, hw, ch), jnp.float32)
                       for ch in (1, 4, 1, 10))
    out_specs = tuple(pl.BlockSpec((1, hwb, ch), lambda i, j: (i, j, 0))
                      for ch in (1, 4, 1, 10))

    cls_p, bbox_p, obj_p, kps_p = pl.pallas_call(
        functools.partial(_yunet_body, H=h, W=w, HWB=hwb),
        out_shape=out_shapes,
        grid=(n, chunks),
        in_specs=in_specs,
        out_specs=out_specs,
        scratch_shapes=[pltpu.VMEM((16, hw), jnp.float32),
                        pltpu.VMEM((9 * c, hw), jnp.float32)],
        compiler_params=pltpu.CompilerParams(
            dimension_semantics=("parallel", "arbitrary")),
    )(*operands)
    return cls_p, bbox_p, obj_p, kps_p


# speedup vs baseline: 13.0582x; 13.0582x over previous
import functools

import jax
import jax.numpy as jnp
from jax.experimental import pallas as pl
from jax.experimental.pallas import tpu as pltpu


def _yunet_body(x_ref, w1_ref, b1_ref, w2_ref, b2_ref, w3_ref, b3_ref,
                wh_ref, bh_ref,
                cls_ref, bbox_ref, obj_ref, kps_ref,
                col_ref, *, H, W, BC):
    HW = H * W
    B = BC // 3

    lane = jax.lax.broadcasted_iota(jnp.int32, (1, HW), 1)
    row = lane // W
    col = lane % W

    taps = []
    for di in range(3):
        for dj in range(3):
            s = (di - 1) * W + (dj - 1)
            amount = (-s) % HW
            valid = ((row + (di - 1) >= 0) & (row + (di - 1) < H) &
                     (col + (dj - 1) >= 0) & (col + (dj - 1) < W))
            taps.append((amount, valid))

    def conv3x3_relu(x, w_ref, b_ref):
        for t, (amount, valid) in enumerate(taps):
            shifted = x if amount == 0 else pltpu.roll(x, amount, axis=1)
            col_ref[t * BC:(t + 1) * BC, :] = jnp.where(valid, shifted, 0.0)
        y = jnp.dot(w_ref[...], col_ref[...],
                    preferred_element_type=jnp.float32) + b_ref[...]
        return jnp.maximum(y, 0.0)

    x = x_ref[...]
    x = conv3x3_relu(x, w1_ref, b1_ref)
    x = conv3x3_relu(x, w2_ref, b2_ref)
    f = conv3x3_relu(x, w3_ref, b3_ref)

    y = jnp.dot(wh_ref[...], f,
                preferred_element_type=jnp.float32) + bh_ref[...]
    cls_ref[...] = y[0:B]
    bbox_ref[...] = y[B:5 * B]
    obj_ref[...] = y[5 * B:6 * B]
    kps_ref[...] = y[6 * B:16 * B]


def kernel(img, dn_w, dn_b, lle_w, lle_b, bb_w, bb_b, hd_w, hd_b):
    n, c, h, w = img.shape
    hw = h * w
    B = 8 if n % 8 == 0 else 1
    eye = jnp.eye(B, dtype=jnp.float32)

    x = img.astype(jnp.float32).reshape(n * c, hw)

    def conv_w_big(wt):
        o, i = wt.shape[0], wt.shape[1]
        wr = jnp.transpose(wt, (2, 3, 0, 1)).reshape(9, o, i)
        big = jnp.einsum('ab,toc->aotbc', eye, wr)
        return big.reshape(B * o, 9 * B * i)

    def conv_b_big(bt):
        return jnp.tile(bt, B).reshape(-1, 1)

    wh_big = jnp.einsum('ab,oc->aobc', eye, hd_w).reshape(B * 16, B * 16)
    bh_big = jnp.tile(hd_b, B).reshape(-1, 1)
    perm = ([b * 16 + 0 for b in range(B)] +
            [b * 16 + 1 + i for b in range(B) for i in range(4)] +
            [b * 16 + 5 for b in range(B)] +
            [b * 16 + 6 + i for b in range(B) for i in range(10)])
    perm = jnp.asarray(perm, dtype=jnp.int32)
    wh_big = wh_big[perm]
    bh_big = bh_big[perm]

    operands = (
        x,
        conv_w_big(dn_w), conv_b_big(dn_b),
        conv_w_big(lle_w), conv_b_big(lle_b),
        conv_w_big(bb_w), conv_b_big(bb_b),
        wh_big, bh_big,
    )
    in_specs = [pl.BlockSpec((B * c, hw), lambda i: (i, 0))]
    in_specs += [pl.BlockSpec(op.shape, lambda i: (0, 0))
                 for op in operands[1:]]

    out_shapes = (
        jax.ShapeDtypeStruct((n, hw), jnp.float32),
        jax.ShapeDtypeStruct((n * 4, hw), jnp.float32),
        jax.ShapeDtypeStruct((n, hw), jnp.float32),
        jax.ShapeDtypeStruct((n * 10, hw), jnp.float32),
    )
    out_specs = (
        pl.BlockSpec((B, hw), lambda i: (i, 0)),
        pl.BlockSpec((B * 4, hw), lambda i: (i, 0)),
        pl.BlockSpec((B, hw), lambda i: (i, 0)),
        pl.BlockSpec((B * 10, hw), lambda i: (i, 0)),
    )

    cls2, bbox2, obj2, kps2 = pl.pallas_call(
        functools.partial(_yunet_body, H=h, W=w, BC=B * c),
        out_shape=out_shapes,
        grid=(n // B,),
        in_specs=in_specs,
        out_specs=out_specs,
        scratch_shapes=[pltpu.VMEM((9 * B * c, hw), jnp.float32)],
        compiler_params=pltpu.CompilerParams(
            dimension_semantics=("parallel",)),
    )(*operands)

    cls_p = cls2.reshape(n, hw, 1)
    bbox_p = jnp.transpose(bbox2.reshape(n, 4, hw), (0, 2, 1))
    obj_p = obj2.reshape(n, hw, 1)
    kps_p = jnp.transpose(kps2.reshape(n, 10, hw), (0, 2, 1))
    return cls_p, bbox_p, obj_p, kps_p


# speedup vs baseline: 13.4293x; 1.0284x over previous
import functools

import jax
import jax.numpy as jnp
from jax.experimental import pallas as pl
from jax.experimental.pallas import tpu as pltpu


def _yunet_body(x_ref, w1_ref, b1_ref, w2_ref, b2_ref, w3_ref, b3_ref,
                wh_ref, bh_ref,
                cls_ref, bbox_ref, obj_ref, kps_ref,
                col_ref, *, H, W, BC):
    HW = H * W
    B = BC // 3

    lane = jax.lax.broadcasted_iota(jnp.int32, (1, HW), 1)
    row = lane // W
    col = lane % W

    taps = []
    for di in range(3):
        for dj in range(3):
            s = (di - 1) * W + (dj - 1)
            amount = (-s) % HW
            valid = ((row + (di - 1) >= 0) & (row + (di - 1) < H) &
                     (col + (dj - 1) >= 0) & (col + (dj - 1) < W))
            taps.append((amount, valid))

    def conv3x3_relu(x, w_ref, b_ref):
        for t, (amount, valid) in enumerate(taps):
            shifted = x if amount == 0 else pltpu.roll(x, amount, axis=1)
            col_ref[t * BC:(t + 1) * BC, :] = jnp.where(
                valid, shifted, 0.0).astype(jnp.bfloat16)
        y = jnp.dot(w_ref[...], col_ref[...],
                    preferred_element_type=jnp.float32) + b_ref[...]
        return jnp.maximum(y, 0.0)

    x = x_ref[...]
    x = conv3x3_relu(x, w1_ref, b1_ref)
    x = conv3x3_relu(x, w2_ref, b2_ref)
    f = conv3x3_relu(x, w3_ref, b3_ref)

    y = jnp.dot(wh_ref[...], f.astype(jnp.bfloat16),
                preferred_element_type=jnp.float32) + bh_ref[...]
    cls_ref[...] = y[0:B].reshape(B * H, W)
    bbox_ref[...] = y[B:5 * B]
    obj_ref[...] = y[5 * B:6 * B].reshape(B * H, W)
    kps_ref[...] = y[6 * B:16 * B]


def kernel(img, dn_w, dn_b, lle_w, lle_b, bb_w, bb_b, hd_w, hd_b):
    n, c, h, w = img.shape
    hw = h * w
    B = 8 if n % 8 == 0 else 1
    eye = jnp.eye(B, dtype=jnp.float32)

    x = img.astype(jnp.float32).reshape(n * c, hw)

    def conv_w_big(wt):
        o, i = wt.shape[0], wt.shape[1]
        wr = jnp.transpose(wt, (2, 3, 0, 1)).reshape(9, o, i)
        big = jnp.einsum('ab,toc->aotbc', eye, wr)
        return big.reshape(B * o, 9 * B * i)

    def conv_b_big(bt):
        return jnp.tile(bt, B).reshape(-1, 1)

    wh_big = jnp.einsum('ab,oc->aobc', eye, hd_w).reshape(B * 16, B * 16)
    bh_big = jnp.tile(hd_b, B).reshape(-1, 1)
    perm = ([b * 16 + 0 for b in range(B)] +
            [b * 16 + 1 + i for b in range(B) for i in range(4)] +
            [b * 16 + 5 for b in range(B)] +
            [b * 16 + 6 + i for b in range(B) for i in range(10)])
    perm = jnp.asarray(perm, dtype=jnp.int32)
    wh_big = wh_big[perm]
    bh_big = bh_big[perm]

    operands = (
        x,
        conv_w_big(dn_w).astype(jnp.bfloat16), conv_b_big(dn_b),
        conv_w_big(lle_w).astype(jnp.bfloat16), conv_b_big(lle_b),
        conv_w_big(bb_w).astype(jnp.bfloat16), conv_b_big(bb_b),
        wh_big.astype(jnp.bfloat16), bh_big,
    )
    in_specs = [pl.BlockSpec((B * c, hw), lambda i: (i, 0))]
    in_specs += [pl.BlockSpec(op.shape, lambda i: (0, 0))
                 for op in operands[1:]]

    out_shapes = (
        jax.ShapeDtypeStruct((n * h, w), jnp.float32),
        jax.ShapeDtypeStruct((n * 4, hw), jnp.float32),
        jax.ShapeDtypeStruct((n * h, w), jnp.float32),
        jax.ShapeDtypeStruct((n * 10, hw), jnp.float32),
    )
    out_specs = (
        pl.BlockSpec((B * h, w), lambda i: (i, 0)),
        pl.BlockSpec((B * 4, hw), lambda i: (i, 0)),
        pl.BlockSpec((B * h, w), lambda i: (i, 0)),
        pl.BlockSpec((B * 10, hw), lambda i: (i, 0)),
    )

    cls2, bbox2, obj2, kps2 = pl.pallas_call(
        functools.partial(_yunet_body, H=h, W=w, BC=B * c),
        out_shape=out_shapes,
        grid=(n // B,),
        in_specs=in_specs,
        out_specs=out_specs,
        scratch_shapes=[pltpu.VMEM((9 * B * c, hw), jnp.bfloat16)],
        compiler_params=pltpu.CompilerParams(
            dimension_semantics=("parallel",)),
    )(*operands)

    cls_p = cls2.reshape(n, hw, 1)
    bbox_p = jnp.transpose(bbox2.reshape(n, 4, hw), (0, 2, 1))
    obj_p = obj2.reshape(n, hw, 1)
    kps_p = jnp.transpose(kps2.reshape(n, 10, hw), (0, 2, 1))
    return cls_p, bbox_p, obj_p, kps_p
